# Initial kernel scaffold; baseline (speedup 1.0000x reference)
import jax
import jax.numpy as jnp
from jax.experimental import pallas as pl


def kernel(atomic_numbers, positions, edge_index, atom_emb, W1, b1, W2, b2, W3, b3):
    raise NotImplementedError("write your pallas kernel here")


# speedup vs baseline: 89.6846x; 89.6846x over previous
import functools

import jax
import jax.numpy as jnp
import numpy as np
from jax import lax
from jax.experimental import pallas as pl
from jax.experimental.pallas import tpu as pltpu
from jax.experimental.pallas import tpu_sc as plsc

RC = 6.0
ETA = 0.5
NUM_RS = 8
PI = float(np.pi)

COS_C = (0.9999999954837723, -0.4999999182511114, 0.04166651545566957,
         -0.0013887904317784216, 2.477170058325944e-05,
         -2.7094472211869996e-07, 1.7294977163936824e-09)
SIN_C = (0.999999999682044, -0.16666666111981374, 0.00833332307674742,
         -0.0001984060248142518, 2.753708367635486e-06,
         -2.4739295909583705e-08, 1.3646969208297968e-10)


def _horner(t, cs):
    acc = jnp.full_like(t, cs[-1])
    for c in cs[-2::-1]:
        acc = acc * t + c
    return acc


def _rsqrt(ss):
    ii = plsc.bitcast(ss, jnp.int32)
    ii = jnp.int32(0x5F3759DF) - lax.shift_right_logical(ii, 1)
    y = plsc.bitcast(ii, jnp.float32)
    for _ in range(3):
        y = y * (1.5 - 0.5 * ss * y * y)
    return y


def _zero_vmem(ref, nwords):
    zf = jnp.zeros((16,), jnp.float32)

    def body(j, c):
        ref[pl.ds(j * 16, 16)] = zf
        return c

    lax.fori_loop(0, nwords // 16, body, 0)


@functools.partial(jax.jit, static_argnums=(3, 4, 5, 6))
def _sc_forward(pos_flat, src, dst, n, e, nw, chunk):
    rs = [RC * i / (NUM_RS - 1) for i in range(NUM_RS)]
    e_per_w = e // nw
    nchunks = e_per_w // chunk

    def body(pos_hbm, src_hbm, dst_hbm, rad_out, d_out, ux_out, uy_out,
             uz_out, pos_v, rad_v, src_v, dst_v, d_v, ux_v, uy_v, uz_v):
        wid = lax.axis_index("s") * 2 + lax.axis_index("c")
        pltpu.sync_copy(pos_hbm, pos_v)
        _zero_vmem(rad_v, NUM_RS * n)
        ebase = wid * e_per_w
        for ci in range(nchunks):
            base = ebase + ci * chunk
            pltpu.sync_copy(src_hbm.at[pl.ds(base, chunk)], src_v)
            pltpu.sync_copy(dst_hbm.at[pl.ds(base, chunk)], dst_v)

            def ebody(j, c):
                off = j * 16
                s = src_v[pl.ds(off, 16)]
                t_ = dst_v[pl.ds(off, 16)]
                s3 = s * 3
                t3 = t_ * 3
                xs = plsc.load_gather(pos_v, [s3])
                ys = plsc.load_gather(pos_v, [s3 + 1])
                zs = plsc.load_gather(pos_v, [s3 + 2])
                xd = plsc.load_gather(pos_v, [t3])
                yd = plsc.load_gather(pos_v, [t3 + 1])
                zd = plsc.load_gather(pos_v, [t3 + 2])
                vx = xd - xs
                vy = yd - ys
                vz = zd - zs
                ss = vx * vx + vy * vy + vz * vz + 1e-12
                r = _rsqrt(ss)
                d = ss * r
                x = d * (PI / RC)
                xc = jnp.minimum(x, PI)
                tt = xc * xc
                cosx = _horner(tt, COS_C)
                fc = jnp.where(d < RC, 0.5 * (cosx + 1.0), 0.0)
                s8 = s * NUM_RS
                for i in range(NUM_RS):
                    dt = d - rs[i]
                    g = jnp.exp((-ETA) * dt * dt) * fc
                    plsc.addupdate_scatter(rad_v, [s8 + i], g)
                d_v[pl.ds(off, 16)] = d
                ux_v[pl.ds(off, 16)] = vx * r
                uy_v[pl.ds(off, 16)] = vy * r
                uz_v[pl.ds(off, 16)] = vz * r
                return c

            lax.fori_loop(0, chunk // 16, ebody, 0)
            pltpu.sync_copy(d_v, d_out.at[pl.ds(base, chunk)])
            pltpu.sync_copy(ux_v, ux_out.at[pl.ds(base, chunk)])
            pltpu.sync_copy(uy_v, uy_out.at[pl.ds(base, chunk)])
            pltpu.sync_copy(uz_v, uz_out.at[pl.ds(base, chunk)])
        pltpu.sync_copy(rad_v, rad_out.at[wid])

    f32 = jnp.float32
    fwd = pl.kernel(
        body,
        out_type=[
            jax.ShapeDtypeStruct((nw, NUM_RS * n), f32),
            jax.ShapeDtypeStruct((e,), f32),
            jax.ShapeDtypeStruct((e,), f32),
            jax.ShapeDtypeStruct((e,), f32),
            jax.ShapeDtypeStruct((e,), f32),
        ],
        mesh=plsc.VectorSubcoreMesh(core_axis_name="c", subcore_axis_name="s"),
        compiler_params=pltpu.CompilerParams(needs_layout_passes=False),
        scratch_types=[
            pltpu.VMEM((3 * n,), f32),
            pltpu.VMEM((NUM_RS * n,), f32),
            pltpu.VMEM((chunk,), jnp.int32),
            pltpu.VMEM((chunk,), jnp.int32),
            pltpu.VMEM((chunk,), f32),
            pltpu.VMEM((chunk,), f32),
            pltpu.VMEM((chunk,), f32),
            pltpu.VMEM((chunk,), f32),
        ],
    )
    return fwd(pos_flat, src, dst)


@functools.partial(jax.jit, static_argnums=(7, 8, 9, 10))
def _sc_backward(src, dst, d_e, ux, uy, uz, q_flat, n, e, nw, chunk):
    rs = [RC * i / (NUM_RS - 1) for i in range(NUM_RS)]
    e_per_w = e // nw
    nchunks = e_per_w // chunk

    def body(src_hbm, dst_hbm, d_hbm, ux_hbm, uy_hbm, uz_hbm, q_hbm, f_out,
             q_v, f_v, src_v, dst_v, d_v, ux_v, uy_v, uz_v):
        wid = lax.axis_index("s") * 2 + lax.axis_index("c")
        pltpu.sync_copy(q_hbm, q_v)
        _zero_vmem(f_v, 3 * n)
        ebase = wid * e_per_w
        for ci in range(nchunks):
            base = ebase + ci * chunk
            pltpu.sync_copy(src_hbm.at[pl.ds(base, chunk)], src_v)
            pltpu.sync_copy(dst_hbm.at[pl.ds(base, chunk)], dst_v)
            pltpu.sync_copy(d_hbm.at[pl.ds(base, chunk)], d_v)
            pltpu.sync_copy(ux_hbm.at[pl.ds(base, chunk)], ux_v)
            pltpu.sync_copy(uy_hbm.at[pl.ds(base, chunk)], uy_v)
            pltpu.sync_copy(uz_hbm.at[pl.ds(base, chunk)], uz_v)

            def ebody(j, c):
                off = j * 16
                s = src_v[pl.ds(off, 16)]
                t_ = dst_v[pl.ds(off, 16)]
                d = d_v[pl.ds(off, 16)]
                x = d * (PI / RC)
                xc = jnp.minimum(x, PI)
                tt = xc * xc
                cosx = _horner(tt, COS_C)
                sinx = xc * _horner(tt, SIN_C)
                inside = d < RC
                fc = jnp.where(inside, 0.5 * (cosx + 1.0), 0.0)
                fcp = jnp.where(inside, (-0.5 * PI / RC) * sinx, 0.0)
                acc = jnp.zeros((16,), jnp.float32)
                s8 = s * NUM_RS
                for i in range(NUM_RS):
                    dt = d - rs[i]
                    ei = jnp.exp((-ETA) * dt * dt)
                    gp = ei * (fcp - (2.0 * ETA) * dt * fc)
                    qi = plsc.load_gather(q_v, [s8 + i])
                    acc = acc + qi * gp
                wx = acc * ux_v[pl.ds(off, 16)]
                wy = acc * uy_v[pl.ds(off, 16)]
                wz = acc * uz_v[pl.ds(off, 16)]
                plsc.addupdate_scatter(f_v, [s], wx)
                plsc.addupdate_scatter(f_v, [s + n], wy)
                plsc.addupdate_scatter(f_v, [s + 2 * n], wz)
                plsc.addupdate_scatter(f_v, [t_], -wx)
                plsc.addupdate_scatter(f_v, [t_ + n], -wy)
                plsc.addupdate_scatter(f_v, [t_ + 2 * n], -wz)
                return c

            lax.fori_loop(0, chunk // 16, ebody, 0)
        pltpu.sync_copy(f_v, f_out.at[wid])

    f32 = jnp.float32
    bwd = pl.kernel(
        body,
        out_type=jax.ShapeDtypeStruct((nw, 3 * n), f32),
        mesh=plsc.VectorSubcoreMesh(core_axis_name="c", subcore_axis_name="s"),
        compiler_params=pltpu.CompilerParams(needs_layout_passes=False),
        scratch_types=[
            pltpu.VMEM((NUM_RS * n,), f32),
            pltpu.VMEM((3 * n,), f32),
            pltpu.VMEM((chunk,), jnp.int32),
            pltpu.VMEM((chunk,), jnp.int32),
            pltpu.VMEM((chunk,), f32),
            pltpu.VMEM((chunk,), f32),
            pltpu.VMEM((chunk,), f32),
            pltpu.VMEM((chunk,), f32),
        ],
    )
    return bwd(src, dst, d_e, ux, uy, uz, q_flat)


def _tc_reduce2d(parts):
    def body(fp, o):
        o[...] = jnp.sum(fp[...], axis=0, keepdims=True)

    nw, m = parts.shape
    return pl.pallas_call(
        body,
        out_shape=jax.ShapeDtypeStruct((1, m), jnp.float32),
    )(parts)


def _tc_dense(rad, emb0, W1, b1, W2, b2, w3r, w3c, b3, n, nw, nb):
    dn = (((1,), (1,)), ((), ()))

    def body(rad_r, e0, w1, b1_, w2, b2_, w3_, w3c_, b3_, ae_o, tot_o, q_o):
        rad = rad_r[...]
        w1a = w1[0:128, :]
        w1r = w1[128:136, :]
        z1a = jnp.dot(e0[...], w1a, preferred_element_type=jnp.float32) + b1_[...]
        z1 = jnp.dot(rad, w1r, preferred_element_type=jnp.float32) + z1a
        s1 = 1.0 / (1.0 + jnp.exp(-z1))
        h1 = z1 * s1
        z2 = jnp.dot(h1, w2[...], preferred_element_type=jnp.float32) + b2_[...]
        s2 = 1.0 / (1.0 + jnp.exp(-z2))
        h2 = z2 * s2
        ae = jnp.dot(h2, w3c_[...],
                     preferred_element_type=jnp.float32) + b3_[0, 0]
        ae_o[...] = ae
        psum = jnp.sum(ae)

        @pl.when(pl.program_id(0) == 0)
        def _init():
            tot_o[0, 0] = psum

        @pl.when(pl.program_id(0) > 0)
        def _acc():
            tot_o[0, 0] = tot_o[0, 0] + psum

        sp2 = s2 * (1.0 + z2 * (1.0 - s2))
        t2 = sp2 * w3_[...]
        dh1 = lax.dot_general(t2, w2[...], dn,
                              preferred_element_type=jnp.float32)
        sp1 = s1 * (1.0 + z1 * (1.0 - s1))
        dz1 = dh1 * sp1
        q_o[...] = lax.dot_general(dz1, w1r, dn,
                                   preferred_element_type=jnp.float32)

    f32 = jnp.float32
    full = lambda shape: pl.BlockSpec(shape, lambda g: (0,) * len(shape))
    return pl.pallas_call(
        body,
        grid=(n // nb,),
        in_specs=[
            pl.BlockSpec((nb, NUM_RS), lambda g: (g, 0)),
            full((1, 128)),
            full((136, 128)),
            full((1, 128)),
            full((128, 128)),
            full((1, 128)),
            full((1, 128)),
            full((128, 1)),
            pl.BlockSpec(memory_space=pltpu.SMEM),
        ],
        out_specs=[
            pl.BlockSpec((nb, 1), lambda g: (g, 0)),
            pl.BlockSpec(memory_space=pltpu.SMEM),
            pl.BlockSpec((nb, NUM_RS), lambda g: (g, 0)),
        ],
        out_shape=[
            jax.ShapeDtypeStruct((n, 1), f32),
            jax.ShapeDtypeStruct((1, 1), f32),
            jax.ShapeDtypeStruct((n, NUM_RS), f32),
        ],
    )(rad, emb0, W1, b1, W2, b2, w3r, w3c, b3)


def _tc_reduce(f_parts):
    def body(fp, o):
        o[...] = jnp.sum(fp[...], axis=0)

    nw, three, n = f_parts.shape
    return pl.pallas_call(
        body,
        out_shape=jax.ShapeDtypeStruct((three, n), jnp.float32),
    )(f_parts)


def kernel(atomic_numbers, positions, edge_index, atom_emb, W1, b1, W2, b2,
           W3, b3):
    n = positions.shape[0]
    e = edge_index.shape[1]
    info = plsc.get_sparse_core_info()
    nw = info.num_cores * info.num_subcores
    chunk = 2000
    pos_flat = positions.reshape(-1)
    src = edge_index[0]
    dst = edge_index[1]
    rad_parts, d_e, ux, uy, uz = _sc_forward(pos_flat, src, dst, n, e, nw,
                                             chunk)
    rad = _tc_reduce2d(rad_parts).reshape(n, NUM_RS)
    ae_col, tot, q_ne = _tc_dense(
        rad, atom_emb[0:1], W1,
        b1.reshape(1, -1), W2, b2.reshape(1, -1), W3.reshape(1, -1), W3,
        b3.reshape(1, 1), n, nw, 2000)
    f_parts = _sc_backward(src, dst, d_e, ux, uy, uz, q_ne.reshape(-1), n, e,
                           nw, chunk)
    forces3 = _tc_reduce(f_parts.reshape(nw, 3, n))
    return (tot[0, 0], forces3.T, ae_col.reshape(-1))


# speedup vs baseline: 98.3424x; 1.0965x over previous
import functools

import jax
import jax.numpy as jnp
import numpy as np
from jax import lax
from jax.experimental import pallas as pl
from jax.experimental.pallas import tpu as pltpu
from jax.experimental.pallas import tpu_sc as plsc

RC = 6.0
ETA = 0.5
NUM_RS = 8
PI = float(np.pi)
UNROLL = 5

COS_C = (0.9999999954837723, -0.4999999182511114, 0.04166651545566957,
         -0.0013887904317784216, 2.477170058325944e-05,
         -2.7094472211869996e-07, 1.7294977163936824e-09)
SIN_C = (0.999999999682044, -0.16666666111981374, 0.00833332307674742,
         -0.0001984060248142518, 2.753708367635486e-06,
         -2.4739295909583705e-08, 1.3646969208297968e-10)


def _horner(t, cs):
    acc = jnp.full_like(t, cs[-1])
    for c in cs[-2::-1]:
        acc = acc * t + c
    return acc


def _rsqrt(ss):
    ii = plsc.bitcast(ss, jnp.int32)
    ii = jnp.int32(0x5F3759DF) - lax.shift_right_logical(ii, 1)
    y = plsc.bitcast(ii, jnp.float32)
    for _ in range(3):
        y = y * (1.5 - 0.5 * ss * y * y)
    return y


def _zero_vmem(ref, nwords, unroll):
    zf = jnp.zeros((16,), jnp.float32)
    groups = nwords // 16

    def body(j, c):
        for k in range(unroll):
            ref[pl.ds((j * unroll + k) * 16, 16)] = zf
        return c

    lax.fori_loop(0, groups // unroll, body, 0)


@functools.partial(jax.jit, static_argnums=(3, 4, 5, 6))
def _sc_forward(pos_flat, src, dst, n, e, nw, chunk):
    rs = [RC * i / (NUM_RS - 1) for i in range(NUM_RS)]
    e_per_w = e // nw
    nchunks = e_per_w // chunk

    def body(pos_hbm, src_hbm, dst_hbm, rad_out, d_out, ux_out, uy_out,
             uz_out, pos_v, rad_v, src_v, dst_v, d_v, ux_v, uy_v, uz_v):
        wid = lax.axis_index("s") * 2 + lax.axis_index("c")
        pltpu.sync_copy(pos_hbm, pos_v)
        _zero_vmem(rad_v, NUM_RS * n, 8)
        ebase = wid * e_per_w
        for ci in range(nchunks):
            base = ebase + ci * chunk
            pltpu.sync_copy(src_hbm.at[pl.ds(base, chunk)], src_v)
            pltpu.sync_copy(dst_hbm.at[pl.ds(base, chunk)], dst_v)

            def egroup(off):
                s = src_v[pl.ds(off, 16)]
                t_ = dst_v[pl.ds(off, 16)]
                s3 = s * 3
                t3 = t_ * 3
                xs = plsc.load_gather(pos_v, [s3])
                ys = plsc.load_gather(pos_v, [s3 + 1])
                zs = plsc.load_gather(pos_v, [s3 + 2])
                xd = plsc.load_gather(pos_v, [t3])
                yd = plsc.load_gather(pos_v, [t3 + 1])
                zd = plsc.load_gather(pos_v, [t3 + 2])
                vx = xd - xs
                vy = yd - ys
                vz = zd - zs
                ss = vx * vx + vy * vy + vz * vz + 1e-12
                r = _rsqrt(ss)
                d = ss * r
                x = d * (PI / RC)
                xc = jnp.minimum(x, PI)
                tt = xc * xc
                cosx = _horner(tt, COS_C)
                fc = jnp.where(d < RC, 0.5 * (cosx + 1.0), 0.0)
                s8 = s * NUM_RS
                for i in range(NUM_RS):
                    dt = d - rs[i]
                    g = jnp.exp((-ETA) * dt * dt) * fc
                    plsc.addupdate_scatter(rad_v, [s8 + i], g)
                d_v[pl.ds(off, 16)] = d
                ux_v[pl.ds(off, 16)] = vx * r
                uy_v[pl.ds(off, 16)] = vy * r
                uz_v[pl.ds(off, 16)] = vz * r

            def ebody(j, c):
                for k in range(UNROLL):
                    egroup((j * UNROLL + k) * 16)
                return c

            lax.fori_loop(0, chunk // (16 * UNROLL), ebody, 0)
            pltpu.sync_copy(d_v, d_out.at[pl.ds(base, chunk)])
            pltpu.sync_copy(ux_v, ux_out.at[pl.ds(base, chunk)])
            pltpu.sync_copy(uy_v, uy_out.at[pl.ds(base, chunk)])
            pltpu.sync_copy(uz_v, uz_out.at[pl.ds(base, chunk)])
        pltpu.sync_copy(rad_v, rad_out.at[wid])

    f32 = jnp.float32
    fwd = pl.kernel(
        body,
        out_type=[
            jax.ShapeDtypeStruct((nw, NUM_RS * n), f32),
            jax.ShapeDtypeStruct((e,), f32),
            jax.ShapeDtypeStruct((e,), f32),
            jax.ShapeDtypeStruct((e,), f32),
            jax.ShapeDtypeStruct((e,), f32),
        ],
        mesh=plsc.VectorSubcoreMesh(core_axis_name="c", subcore_axis_name="s"),
        compiler_params=pltpu.CompilerParams(needs_layout_passes=False),
        scratch_types=[
            pltpu.VMEM((3 * n,), f32),
            pltpu.VMEM((NUM_RS * n,), f32),
            pltpu.VMEM((chunk,), jnp.int32),
            pltpu.VMEM((chunk,), jnp.int32),
            pltpu.VMEM((chunk,), f32),
            pltpu.VMEM((chunk,), f32),
            pltpu.VMEM((chunk,), f32),
            pltpu.VMEM((chunk,), f32),
        ],
    )
    return fwd(pos_flat, src, dst)


@functools.partial(jax.jit, static_argnums=(7, 8, 9, 10))
def _sc_backward(src, dst, d_e, ux, uy, uz, q_flat, n, e, nw, chunk):
    rs = [RC * i / (NUM_RS - 1) for i in range(NUM_RS)]
    e_per_w = e // nw
    nchunks = e_per_w // chunk

    def body(src_hbm, dst_hbm, d_hbm, ux_hbm, uy_hbm, uz_hbm, q_hbm, f_out,
             q_v, f_v, src_v, dst_v, d_v, ux_v, uy_v, uz_v):
        wid = lax.axis_index("s") * 2 + lax.axis_index("c")
        pltpu.sync_copy(q_hbm, q_v)
        _zero_vmem(f_v, 3 * n, 5)
        ebase = wid * e_per_w
        for ci in range(nchunks):
            base = ebase + ci * chunk
            pltpu.sync_copy(src_hbm.at[pl.ds(base, chunk)], src_v)
            pltpu.sync_copy(dst_hbm.at[pl.ds(base, chunk)], dst_v)
            pltpu.sync_copy(d_hbm.at[pl.ds(base, chunk)], d_v)
            pltpu.sync_copy(ux_hbm.at[pl.ds(base, chunk)], ux_v)
            pltpu.sync_copy(uy_hbm.at[pl.ds(base, chunk)], uy_v)
            pltpu.sync_copy(uz_hbm.at[pl.ds(base, chunk)], uz_v)

            def egroup(off):
                s = src_v[pl.ds(off, 16)]
                t_ = dst_v[pl.ds(off, 16)]
                d = d_v[pl.ds(off, 16)]
                x = d * (PI / RC)
                xc = jnp.minimum(x, PI)
                tt = xc * xc
                cosx = _horner(tt, COS_C)
                sinx = xc * _horner(tt, SIN_C)
                inside = d < RC
                fc = jnp.where(inside, 0.5 * (cosx + 1.0), 0.0)
                fcp = jnp.where(inside, (-0.5 * PI / RC) * sinx, 0.0)
                acc = jnp.zeros((16,), jnp.float32)
                s8 = s * NUM_RS
                for i in range(NUM_RS):
                    dt = d - rs[i]
                    ei = jnp.exp((-ETA) * dt * dt)
                    gp = ei * (fcp - (2.0 * ETA) * dt * fc)
                    qi = plsc.load_gather(q_v, [s8 + i])
                    acc = acc + qi * gp
                wx = acc * ux_v[pl.ds(off, 16)]
                wy = acc * uy_v[pl.ds(off, 16)]
                wz = acc * uz_v[pl.ds(off, 16)]
                plsc.addupdate_scatter(f_v, [s], wx)
                plsc.addupdate_scatter(f_v, [s + n], wy)
                plsc.addupdate_scatter(f_v, [s + 2 * n], wz)
                plsc.addupdate_scatter(f_v, [t_], -wx)
                plsc.addupdate_scatter(f_v, [t_ + n], -wy)
                plsc.addupdate_scatter(f_v, [t_ + 2 * n], -wz)

            def ebody(j, c):
                for k in range(UNROLL):
                    egroup((j * UNROLL + k) * 16)
                return c

            lax.fori_loop(0, chunk // (16 * UNROLL), ebody, 0)
        pltpu.sync_copy(f_v, f_out.at[wid])

    f32 = jnp.float32
    bwd = pl.kernel(
        body,
        out_type=jax.ShapeDtypeStruct((nw, 3 * n), f32),
        mesh=plsc.VectorSubcoreMesh(core_axis_name="c", subcore_axis_name="s"),
        compiler_params=pltpu.CompilerParams(needs_layout_passes=False),
        scratch_types=[
            pltpu.VMEM((NUM_RS * n,), f32),
            pltpu.VMEM((3 * n,), f32),
            pltpu.VMEM((chunk,), jnp.int32),
            pltpu.VMEM((chunk,), jnp.int32),
            pltpu.VMEM((chunk,), f32),
            pltpu.VMEM((chunk,), f32),
            pltpu.VMEM((chunk,), f32),
            pltpu.VMEM((chunk,), f32),
        ],
    )
    return bwd(src, dst, d_e, ux, uy, uz, q_flat)


def _tc_reduce2d(parts):
    def body(fp, o):
        o[...] = jnp.sum(fp[...], axis=0, keepdims=True)

    nw, m = parts.shape
    return pl.pallas_call(
        body,
        out_shape=jax.ShapeDtypeStruct((1, m), jnp.float32),
    )(parts)


def _tc_dense(rad, emb0, W1, b1, W2, b2, w3r, w3c, b3, n, nw, nb):
    dn = (((1,), (1,)), ((), ()))

    def body(rad_r, e0, w1, b1_, w2, b2_, w3_, w3c_, b3_, ae_o, tot_o, q_o):
        rad = rad_r[...]
        w1a = w1[0:128, :]
        w1r = w1[128:136, :]
        z1a = jnp.dot(e0[...], w1a, preferred_element_type=jnp.float32) + b1_[...]
        z1 = jnp.dot(rad, w1r, preferred_element_type=jnp.float32) + z1a
        s1 = 1.0 / (1.0 + jnp.exp(-z1))
        h1 = z1 * s1
        z2 = jnp.dot(h1, w2[...], preferred_element_type=jnp.float32) + b2_[...]
        s2 = 1.0 / (1.0 + jnp.exp(-z2))
        h2 = z2 * s2
        ae = jnp.dot(h2, w3c_[...],
                     preferred_element_type=jnp.float32) + b3_[0, 0]
        ae_o[...] = ae
        psum = jnp.sum(ae)

        @pl.when(pl.program_id(0) == 0)
        def _init():
            tot_o[0, 0] = psum

        @pl.when(pl.program_id(0) > 0)
        def _acc():
            tot_o[0, 0] = tot_o[0, 0] + psum

        sp2 = s2 * (1.0 + z2 * (1.0 - s2))
        t2 = sp2 * w3_[...]
        dh1 = lax.dot_general(t2, w2[...], dn,
                              preferred_element_type=jnp.float32)
        sp1 = s1 * (1.0 + z1 * (1.0 - s1))
        dz1 = dh1 * sp1
        q_o[...] = lax.dot_general(dz1, w1r, dn,
                                   preferred_element_type=jnp.float32)

    f32 = jnp.float32
    full = lambda shape: pl.BlockSpec(shape, lambda g: (0,) * len(shape))
    return pl.pallas_call(
        body,
        grid=(n // nb,),
        in_specs=[
            pl.BlockSpec((nb, NUM_RS), lambda g: (g, 0)),
            full((1, 128)),
            full((136, 128)),
            full((1, 128)),
            full((128, 128)),
            full((1, 128)),
            full((1, 128)),
            full((128, 1)),
            pl.BlockSpec(memory_space=pltpu.SMEM),
        ],
        out_specs=[
            pl.BlockSpec((nb, 1), lambda g: (g, 0)),
            pl.BlockSpec(memory_space=pltpu.SMEM),
            pl.BlockSpec((nb, NUM_RS), lambda g: (g, 0)),
        ],
        out_shape=[
            jax.ShapeDtypeStruct((n, 1), f32),
            jax.ShapeDtypeStruct((1, 1), f32),
            jax.ShapeDtypeStruct((n, NUM_RS), f32),
        ],
    )(rad, emb0, W1, b1, W2, b2, w3r, w3c, b3)


def _tc_reduce(f_parts):
    def body(fp, o):
        o[...] = jnp.sum(fp[...], axis=0)

    nw, three, n = f_parts.shape
    return pl.pallas_call(
        body,
        out_shape=jax.ShapeDtypeStruct((three, n), jnp.float32),
    )(f_parts)


def kernel(atomic_numbers, positions, edge_index, atom_emb, W1, b1, W2, b2,
           W3, b3):
    n = positions.shape[0]
    e = edge_index.shape[1]
    info = plsc.get_sparse_core_info()
    nw = info.num_cores * info.num_subcores
    chunk = 2000
    pos_flat = positions.reshape(-1)
    src = edge_index[0]
    dst = edge_index[1]
    rad_parts, d_e, ux, uy, uz = _sc_forward(pos_flat, src, dst, n, e, nw,
                                             chunk)
    rad = _tc_reduce2d(rad_parts).reshape(n, NUM_RS)
    ae_col, tot, q_ne = _tc_dense(
        rad, atom_emb[0:1], W1,
        b1.reshape(1, -1), W2, b2.reshape(1, -1), W3.reshape(1, -1), W3,
        b3.reshape(1, 1), n, nw, 2000)
    f_parts = _sc_backward(src, dst, d_e, ux, uy, uz, q_ne.reshape(-1), n, e,
                           nw, chunk)
    forces3 = _tc_reduce(f_parts.reshape(nw, 3, n))
    return (tot[0, 0], forces3.T, ae_col.reshape(-1))


# speedup vs baseline: 111.3663x; 1.1324x over previous
import functools

import jax
import jax.numpy as jnp
import numpy as np
from jax import lax
from jax.experimental import pallas as pl
from jax.experimental.pallas import tpu as pltpu
from jax.experimental.pallas import tpu_sc as plsc

RC = 6.0
ETA = 0.5
NUM_RS = 8
PI = float(np.pi)
UNROLL = 1

COS_C = (0.9999999954837723, -0.4999999182511114, 0.04166651545566957,
         -0.0013887904317784216, 2.477170058325944e-05,
         -2.7094472211869996e-07, 1.7294977163936824e-09)
SIN_C = (0.999999999682044, -0.16666666111981374, 0.00833332307674742,
         -0.0001984060248142518, 2.753708367635486e-06,
         -2.4739295909583705e-08, 1.3646969208297968e-10)


def _horner(t, cs):
    acc = jnp.full_like(t, cs[-1])
    for c in cs[-2::-1]:
        acc = acc * t + c
    return acc


def _rsqrt(ss):
    ii = plsc.bitcast(ss, jnp.int32)
    ii = jnp.int32(0x5F3759DF) - lax.shift_right_logical(ii, 1)
    y = plsc.bitcast(ii, jnp.float32)
    for _ in range(3):
        y = y * (1.5 - 0.5 * ss * y * y)
    return y


def _zero_vmem(ref, nwords, unroll):
    zf = jnp.zeros((16,), jnp.float32)
    groups = nwords // 16

    def body(j, c):
        for k in range(unroll):
            ref[pl.ds((j * unroll + k) * 16, 16)] = zf
        return c

    lax.fori_loop(0, groups // unroll, body, 0)


@functools.partial(jax.jit, static_argnums=(3, 4, 5, 6))
def _sc_forward(pos_flat, src, dst, n, e, nw, chunk):
    rs = [RC * i / (NUM_RS - 1) for i in range(NUM_RS)]
    e_per_w = e // nw
    nchunks = e_per_w // chunk

    def body(pos_hbm, src_hbm, dst_hbm, rad_out, d_out, ux_out, uy_out,
             uz_out, pos_v, rad_v, src_v, dst_v, d_v, ux_v, uy_v, uz_v):
        wid = lax.axis_index("s") * 2 + lax.axis_index("c")
        pltpu.sync_copy(pos_hbm, pos_v)
        _zero_vmem(rad_v, NUM_RS * n, 8)
        ebase = wid * e_per_w
        for ci in range(nchunks):
            base = ebase + ci * chunk
            pltpu.sync_copy(src_hbm.at[pl.ds(base, chunk)], src_v)
            pltpu.sync_copy(dst_hbm.at[pl.ds(base, chunk)], dst_v)

            def egroup(off):
                s = src_v[pl.ds(off, 16)]
                t_ = dst_v[pl.ds(off, 16)]
                s3 = s * 3
                t3 = t_ * 3
                xs = plsc.load_gather(pos_v, [s3])
                ys = plsc.load_gather(pos_v, [s3 + 1])
                zs = plsc.load_gather(pos_v, [s3 + 2])
                xd = plsc.load_gather(pos_v, [t3])
                yd = plsc.load_gather(pos_v, [t3 + 1])
                zd = plsc.load_gather(pos_v, [t3 + 2])
                vx = xd - xs
                vy = yd - ys
                vz = zd - zs
                ss = vx * vx + vy * vy + vz * vz + 1e-12
                r = _rsqrt(ss)
                d = ss * r
                x = d * (PI / RC)
                xc = jnp.minimum(x, PI)
                tt = xc * xc
                cosx = _horner(tt, COS_C)
                inside = d < RC
                fc = jnp.where(inside, 0.5 * (cosx + 1.0), 0.0)
                s8 = s * NUM_RS
                for i in range(NUM_RS):
                    dt = d - rs[i]
                    g = jnp.exp((-ETA) * dt * dt) * fc
                    plsc.addupdate_scatter(rad_v, [s8 + i], g, mask=inside)
                d_v[pl.ds(off, 16)] = d
                ux_v[pl.ds(off, 16)] = vx * r
                uy_v[pl.ds(off, 16)] = vy * r
                uz_v[pl.ds(off, 16)] = vz * r

            def ebody(j, c):
                for k in range(UNROLL):
                    egroup((j * UNROLL + k) * 16)
                return c

            lax.fori_loop(0, chunk // (16 * UNROLL), ebody, 0)
            pltpu.sync_copy(d_v, d_out.at[pl.ds(base, chunk)])
            pltpu.sync_copy(ux_v, ux_out.at[pl.ds(base, chunk)])
            pltpu.sync_copy(uy_v, uy_out.at[pl.ds(base, chunk)])
            pltpu.sync_copy(uz_v, uz_out.at[pl.ds(base, chunk)])
        pltpu.sync_copy(rad_v, rad_out.at[wid])

    f32 = jnp.float32
    fwd = pl.kernel(
        body,
        out_type=[
            jax.ShapeDtypeStruct((nw, NUM_RS * n), f32),
            jax.ShapeDtypeStruct((e,), f32),
            jax.ShapeDtypeStruct((e,), f32),
            jax.ShapeDtypeStruct((e,), f32),
            jax.ShapeDtypeStruct((e,), f32),
        ],
        mesh=plsc.VectorSubcoreMesh(core_axis_name="c", subcore_axis_name="s"),
        compiler_params=pltpu.CompilerParams(needs_layout_passes=False),
        scratch_types=[
            pltpu.VMEM((3 * n,), f32),
            pltpu.VMEM((NUM_RS * n,), f32),
            pltpu.VMEM((chunk,), jnp.int32),
            pltpu.VMEM((chunk,), jnp.int32),
            pltpu.VMEM((chunk,), f32),
            pltpu.VMEM((chunk,), f32),
            pltpu.VMEM((chunk,), f32),
            pltpu.VMEM((chunk,), f32),
        ],
    )
    return fwd(pos_flat, src, dst)


@functools.partial(jax.jit, static_argnums=(7, 8, 9, 10))
def _sc_backward(src, dst, d_e, ux, uy, uz, q_flat, n, e, nw, chunk):
    rs = [RC * i / (NUM_RS - 1) for i in range(NUM_RS)]
    e_per_w = e // nw
    nchunks = e_per_w // chunk

    def body(src_hbm, dst_hbm, d_hbm, ux_hbm, uy_hbm, uz_hbm, q_hbm, f_out,
             q_v, f_v, src_v, dst_v, d_v, ux_v, uy_v, uz_v):
        wid = lax.axis_index("s") * 2 + lax.axis_index("c")
        pltpu.sync_copy(q_hbm, q_v)
        _zero_vmem(f_v, 3 * n, 5)
        ebase = wid * e_per_w
        for ci in range(nchunks):
            base = ebase + ci * chunk
            pltpu.sync_copy(src_hbm.at[pl.ds(base, chunk)], src_v)
            pltpu.sync_copy(dst_hbm.at[pl.ds(base, chunk)], dst_v)
            pltpu.sync_copy(d_hbm.at[pl.ds(base, chunk)], d_v)
            pltpu.sync_copy(ux_hbm.at[pl.ds(base, chunk)], ux_v)
            pltpu.sync_copy(uy_hbm.at[pl.ds(base, chunk)], uy_v)
            pltpu.sync_copy(uz_hbm.at[pl.ds(base, chunk)], uz_v)

            def egroup(off):
                s = src_v[pl.ds(off, 16)]
                t_ = dst_v[pl.ds(off, 16)]
                d = d_v[pl.ds(off, 16)]
                x = d * (PI / RC)
                xc = jnp.minimum(x, PI)
                tt = xc * xc
                cosx = _horner(tt, COS_C)
                sinx = xc * _horner(tt, SIN_C)
                inside = d < RC
                fc = jnp.where(inside, 0.5 * (cosx + 1.0), 0.0)
                fcp = jnp.where(inside, (-0.5 * PI / RC) * sinx, 0.0)
                acc = jnp.zeros((16,), jnp.float32)
                s8 = s * NUM_RS
                for i in range(NUM_RS):
                    dt = d - rs[i]
                    ei = jnp.exp((-ETA) * dt * dt)
                    gp = ei * (fcp - (2.0 * ETA) * dt * fc)
                    qi = plsc.load_gather(q_v, [s8 + i], mask=inside)
                    acc = acc + jnp.where(inside, qi, 0.0) * gp
                wx = acc * ux_v[pl.ds(off, 16)]
                wy = acc * uy_v[pl.ds(off, 16)]
                wz = acc * uz_v[pl.ds(off, 16)]
                plsc.addupdate_scatter(f_v, [s], wx, mask=inside)
                plsc.addupdate_scatter(f_v, [s + n], wy, mask=inside)
                plsc.addupdate_scatter(f_v, [s + 2 * n], wz, mask=inside)
                plsc.addupdate_scatter(f_v, [t_], -wx, mask=inside)
                plsc.addupdate_scatter(f_v, [t_ + n], -wy, mask=inside)
                plsc.addupdate_scatter(f_v, [t_ + 2 * n], -wz, mask=inside)

            def ebody(j, c):
                for k in range(UNROLL):
                    egroup((j * UNROLL + k) * 16)
                return c

            lax.fori_loop(0, chunk // (16 * UNROLL), ebody, 0)
        pltpu.sync_copy(f_v, f_out.at[wid])

    f32 = jnp.float32
    bwd = pl.kernel(
        body,
        out_type=jax.ShapeDtypeStruct((nw, 3 * n), f32),
        mesh=plsc.VectorSubcoreMesh(core_axis_name="c", subcore_axis_name="s"),
        compiler_params=pltpu.CompilerParams(needs_layout_passes=False),
        scratch_types=[
            pltpu.VMEM((NUM_RS * n,), f32),
            pltpu.VMEM((3 * n,), f32),
            pltpu.VMEM((chunk,), jnp.int32),
            pltpu.VMEM((chunk,), jnp.int32),
            pltpu.VMEM((chunk,), f32),
            pltpu.VMEM((chunk,), f32),
            pltpu.VMEM((chunk,), f32),
            pltpu.VMEM((chunk,), f32),
        ],
    )
    return bwd(src, dst, d_e, ux, uy, uz, q_flat)


def _tc_reduce2d(parts):
    def body(fp, o):
        o[...] = jnp.sum(fp[...], axis=0, keepdims=True)

    nw, m = parts.shape
    return pl.pallas_call(
        body,
        out_shape=jax.ShapeDtypeStruct((1, m), jnp.float32),
    )(parts)


def _tc_dense(rad, emb0, W1, b1, W2, b2, w3r, w3c, b3, n, nw, nb):
    dn = (((1,), (1,)), ((), ()))

    def body(rad_r, e0, w1, b1_, w2, b2_, w3_, w3c_, b3_, ae_o, tot_o, q_o):
        rad = rad_r[...]
        w1a = w1[0:128, :]
        w1r = w1[128:136, :]
        z1a = jnp.dot(e0[...], w1a, preferred_element_type=jnp.float32) + b1_[...]
        z1 = jnp.dot(rad, w1r, preferred_element_type=jnp.float32) + z1a
        s1 = 1.0 / (1.0 + jnp.exp(-z1))
        h1 = z1 * s1
        z2 = jnp.dot(h1, w2[...], preferred_element_type=jnp.float32) + b2_[...]
        s2 = 1.0 / (1.0 + jnp.exp(-z2))
        h2 = z2 * s2
        ae = jnp.dot(h2, w3c_[...],
                     preferred_element_type=jnp.float32) + b3_[0, 0]
        ae_o[...] = ae
        psum = jnp.sum(ae)

        @pl.when(pl.program_id(0) == 0)
        def _init():
            tot_o[0, 0] = psum

        @pl.when(pl.program_id(0) > 0)
        def _acc():
            tot_o[0, 0] = tot_o[0, 0] + psum

        sp2 = s2 * (1.0 + z2 * (1.0 - s2))
        t2 = sp2 * w3_[...]
        dh1 = lax.dot_general(t2, w2[...], dn,
                              preferred_element_type=jnp.float32)
        sp1 = s1 * (1.0 + z1 * (1.0 - s1))
        dz1 = dh1 * sp1
        q_o[...] = lax.dot_general(dz1, w1r, dn,
                                   preferred_element_type=jnp.float32)

    f32 = jnp.float32
    full = lambda shape: pl.BlockSpec(shape, lambda g: (0,) * len(shape))
    return pl.pallas_call(
        body,
        grid=(n // nb,),
        in_specs=[
            pl.BlockSpec((nb, NUM_RS), lambda g: (g, 0)),
            full((1, 128)),
            full((136, 128)),
            full((1, 128)),
            full((128, 128)),
            full((1, 128)),
            full((1, 128)),
            full((128, 1)),
            pl.BlockSpec(memory_space=pltpu.SMEM),
        ],
        out_specs=[
            pl.BlockSpec((nb, 1), lambda g: (g, 0)),
            pl.BlockSpec(memory_space=pltpu.SMEM),
            pl.BlockSpec((nb, NUM_RS), lambda g: (g, 0)),
        ],
        out_shape=[
            jax.ShapeDtypeStruct((n, 1), f32),
            jax.ShapeDtypeStruct((1, 1), f32),
            jax.ShapeDtypeStruct((n, NUM_RS), f32),
        ],
    )(rad, emb0, W1, b1, W2, b2, w3r, w3c, b3)


def _tc_reduce(f_parts):
    def body(fp, o):
        o[...] = jnp.sum(fp[...], axis=0)

    nw, three, n = f_parts.shape
    return pl.pallas_call(
        body,
        out_shape=jax.ShapeDtypeStruct((three, n), jnp.float32),
    )(f_parts)


def kernel(atomic_numbers, positions, edge_index, atom_emb, W1, b1, W2, b2,
           W3, b3):
    n = positions.shape[0]
    e = edge_index.shape[1]
    info = plsc.get_sparse_core_info()
    nw = info.num_cores * info.num_subcores
    chunk = 2000
    pos_flat = positions.reshape(-1)
    src = edge_index[0]
    dst = edge_index[1]
    rad_parts, d_e, ux, uy, uz = _sc_forward(pos_flat, src, dst, n, e, nw,
                                             chunk)
    rad = _tc_reduce2d(rad_parts).reshape(n, NUM_RS)
    ae_col, tot, q_ne = _tc_dense(
        rad, atom_emb[0:1], W1,
        b1.reshape(1, -1), W2, b2.reshape(1, -1), W3.reshape(1, -1), W3,
        b3.reshape(1, 1), n, nw, 2000)
    f_parts = _sc_backward(src, dst, d_e, ux, uy, uz, q_ne.reshape(-1), n, e,
                           nw, chunk)
    forces3 = _tc_reduce(f_parts.reshape(nw, 3, n))
    return (tot[0, 0], forces3.T, ae_col.reshape(-1))


# speedup vs baseline: 134.2043x; 1.2051x over previous
import functools

import jax
import jax.numpy as jnp
import numpy as np
from jax import lax
from jax.experimental import pallas as pl
from jax.experimental.pallas import tpu as pltpu
from jax.experimental.pallas import tpu_sc as plsc

RC = 6.0
ETA = 0.5
NUM_RS = 8
PI = float(np.pi)
UNROLL = 1

COS_C = (0.9999999954837723, -0.4999999182511114, 0.04166651545566957,
         -0.0013887904317784216, 2.477170058325944e-05,
         -2.7094472211869996e-07, 1.7294977163936824e-09)
SIN_C = (0.999999999682044, -0.16666666111981374, 0.00833332307674742,
         -0.0001984060248142518, 2.753708367635486e-06,
         -2.4739295909583705e-08, 1.3646969208297968e-10)


def _horner(t, cs):
    acc = jnp.full_like(t, cs[-1])
    for c in cs[-2::-1]:
        acc = acc * t + c
    return acc


def _rsqrt(ss):
    ii = plsc.bitcast(ss, jnp.int32)
    ii = jnp.int32(0x5F3759DF) - lax.shift_right_logical(ii, 1)
    y = plsc.bitcast(ii, jnp.float32)
    for _ in range(3):
        y = y * (1.5 - 0.5 * ss * y * y)
    return y


def _zero_vmem(ref, nwords, unroll):
    zf = jnp.zeros((16,), jnp.float32)
    groups = nwords // 16

    def body(j, c):
        for k in range(unroll):
            ref[pl.ds((j * unroll + k) * 16, 16)] = zf
        return c

    lax.fori_loop(0, groups // unroll, body, 0)


@functools.partial(jax.jit, static_argnums=(3, 4, 5, 6, 7))
def _sc_forward(pos_flat, src, dst, n, e, nw, chunk, cap):
    rs = [RC * i / (NUM_RS - 1) for i in range(NUM_RS)]
    e_per_w = e // nw
    nchunks = e_per_w // chunk

    def body(pos_hbm, src_hbm, dst_hbm, rad_out, cmp_out, cnt_out, pos_v,
             rad_v, src_v, dst_v, cb_v, cnt_v):
        wid = lax.axis_index("s") * 2 + lax.axis_index("c")
        lane = lax.iota(jnp.int32, 16)
        cnts_vec = jnp.zeros((16,), jnp.int32)
        pltpu.sync_copy(pos_hbm, pos_v)
        _zero_vmem(rad_v, NUM_RS * n, 8)
        _zero_vmem(cb_v, 6 * cap, 8)
        ebase = wid * e_per_w
        for ci in range(nchunks):
            base = ebase + ci * chunk
            pltpu.sync_copy(src_hbm.at[pl.ds(base, chunk)], src_v)
            pltpu.sync_copy(dst_hbm.at[pl.ds(base, chunk)], dst_v)

            def ebody(j, cnt):
                off = j * 16
                s = src_v[pl.ds(off, 16)]
                t_ = dst_v[pl.ds(off, 16)]
                s3 = s * 3
                t3 = t_ * 3
                xs = plsc.load_gather(pos_v, [s3])
                ys = plsc.load_gather(pos_v, [s3 + 1])
                zs = plsc.load_gather(pos_v, [s3 + 2])
                xd = plsc.load_gather(pos_v, [t3])
                yd = plsc.load_gather(pos_v, [t3 + 1])
                zd = plsc.load_gather(pos_v, [t3 + 2])
                vx = xd - xs
                vy = yd - ys
                vz = zd - zs
                ss = vx * vx + vy * vy + vz * vz + 1e-12
                r = _rsqrt(ss)
                d = ss * r
                x = d * (PI / RC)
                xc = jnp.minimum(x, PI)
                tt = xc * xc
                cosx = _horner(tt, COS_C)
                inside = d < RC
                fc = jnp.where(inside, 0.5 * (cosx + 1.0), 0.0)
                s8 = s * NUM_RS
                for i in range(NUM_RS):
                    dt = d - rs[i]
                    g = jnp.exp((-ETA) * dt * dt) * fc
                    plsc.addupdate_scatter(rad_v, [s8 + i], g, mask=inside)
                plsc.store_compressed(cb_v.at[pl.ds(cnt, 16)],
                                      plsc.bitcast(s8, jnp.float32),
                                      mask=inside)
                plsc.store_compressed(cb_v.at[pl.ds(cap + cnt, 16)],
                                      plsc.bitcast(t_, jnp.float32),
                                      mask=inside)
                plsc.store_compressed(cb_v.at[pl.ds(2 * cap + cnt, 16)], d,
                                      mask=inside)
                plsc.store_compressed(cb_v.at[pl.ds(3 * cap + cnt, 16)],
                                      vx * r, mask=inside)
                plsc.store_compressed(cb_v.at[pl.ds(4 * cap + cnt, 16)],
                                      vy * r, mask=inside)
                plsc.store_compressed(cb_v.at[pl.ds(5 * cap + cnt, 16)],
                                      vz * r, mask=inside)
                pc = plsc.all_reduce_population_count(inside)
                return jnp.minimum(cnt + pc[0], jnp.int32(cap - 16))

            cnt = lax.fori_loop(0, chunk // 16, ebody, jnp.int32(0))
            cnts_vec = jnp.where(lane == ci, cnt, cnts_vec)
            pltpu.sync_copy(cb_v, cmp_out.at[wid * nchunks + ci])
        cnt_v[...] = cnts_vec
        pltpu.sync_copy(rad_v, rad_out.at[wid])
        pltpu.sync_copy(cnt_v, cnt_out.at[wid])

    f32 = jnp.float32
    fwd = pl.kernel(
        body,
        out_type=[
            jax.ShapeDtypeStruct((nw, NUM_RS * n), f32),
            jax.ShapeDtypeStruct((nw * nchunks, 6 * cap), f32),
            jax.ShapeDtypeStruct((nw, 16), jnp.int32),
        ],
        mesh=plsc.VectorSubcoreMesh(core_axis_name="c", subcore_axis_name="s"),
        compiler_params=pltpu.CompilerParams(needs_layout_passes=False),
        scratch_types=[
            pltpu.VMEM((3 * n,), f32),
            pltpu.VMEM((NUM_RS * n,), f32),
            pltpu.VMEM((chunk,), jnp.int32),
            pltpu.VMEM((chunk,), jnp.int32),
            pltpu.VMEM((6 * cap,), f32),
            pltpu.VMEM((16,), jnp.int32),
        ],
    )
    return fwd(pos_flat, src, dst)


@functools.partial(jax.jit, static_argnums=(3, 4, 5, 6))
def _sc_backward(cmp, cnts, q_flat, n, nw, nchunks, cap):
    rs = [RC * i / (NUM_RS - 1) for i in range(NUM_RS)]

    def body(cmp_hbm, cnt_hbm, q_hbm, f_out, q_v, f_v, cb_v, cnt_v):
        wid = lax.axis_index("s") * 2 + lax.axis_index("c")
        lane = lax.iota(jnp.int32, 16)
        pltpu.sync_copy(q_hbm, q_v)
        pltpu.sync_copy(cnt_hbm.at[wid], cnt_v)
        _zero_vmem(f_v, 3 * n, 5)
        cnts_all = cnt_v[...]
        for ci in range(nchunks):
            pltpu.sync_copy(cmp_hbm.at[wid * nchunks + ci], cb_v)
            cnt = cnts_all[ci]

            def ebody(j, c):
                off = j * 16
                m = lane < (cnt - off)
                s8 = plsc.bitcast(cb_v[pl.ds(off, 16)], jnp.int32)
                t_ = plsc.bitcast(cb_v[pl.ds(cap + off, 16)], jnp.int32)
                s8 = jnp.where(m, s8, 0)
                t_ = jnp.where(m, t_, 0)
                d = cb_v[pl.ds(2 * cap + off, 16)]
                x = d * (PI / RC)
                xc = jnp.minimum(x, PI)
                tt = xc * xc
                cosx = _horner(tt, COS_C)
                sinx = xc * _horner(tt, SIN_C)
                fc = 0.5 * (cosx + 1.0)
                fcp = (-0.5 * PI / RC) * sinx
                acc = jnp.zeros((16,), jnp.float32)
                for i in range(NUM_RS):
                    dt = d - rs[i]
                    ei = jnp.exp((-ETA) * dt * dt)
                    gp = ei * (fcp - (2.0 * ETA) * dt * fc)
                    qi = plsc.load_gather(q_v, [s8 + i], mask=m)
                    acc = acc + jnp.where(m, qi, 0.0) * gp
                wx = acc * cb_v[pl.ds(3 * cap + off, 16)]
                wy = acc * cb_v[pl.ds(4 * cap + off, 16)]
                wz = acc * cb_v[pl.ds(5 * cap + off, 16)]
                s = lax.shift_right_logical(s8, 3)
                plsc.addupdate_scatter(f_v, [s], wx, mask=m)
                plsc.addupdate_scatter(f_v, [s + n], wy, mask=m)
                plsc.addupdate_scatter(f_v, [s + 2 * n], wz, mask=m)
                plsc.addupdate_scatter(f_v, [t_], -wx, mask=m)
                plsc.addupdate_scatter(f_v, [t_ + n], -wy, mask=m)
                plsc.addupdate_scatter(f_v, [t_ + 2 * n], -wz, mask=m)
                return c

            ng = jnp.minimum(lax.div(cnt + 15, 16), jnp.int32(cap // 16))
            ng = jnp.maximum(ng, 0)
            lax.fori_loop(0, ng, ebody, 0)
        pltpu.sync_copy(f_v, f_out.at[wid])

    f32 = jnp.float32
    bwd = pl.kernel(
        body,
        out_type=jax.ShapeDtypeStruct((nw, 3 * n), f32),
        mesh=plsc.VectorSubcoreMesh(core_axis_name="c", subcore_axis_name="s"),
        compiler_params=pltpu.CompilerParams(needs_layout_passes=False),
        scratch_types=[
            pltpu.VMEM((NUM_RS * n,), f32),
            pltpu.VMEM((3 * n,), f32),
            pltpu.VMEM((6 * cap,), f32),
            pltpu.VMEM((16,), jnp.int32),
        ],
    )
    return bwd(cmp, cnts, q_flat)


def _tc_reduce2d(parts):
    def body(fp, o):
        o[...] = jnp.sum(fp[...], axis=0, keepdims=True)

    nw, m = parts.shape
    return pl.pallas_call(
        body,
        out_shape=jax.ShapeDtypeStruct((1, m), jnp.float32),
    )(parts)


def _tc_dense(rad, emb0, W1, b1, W2, b2, w3r, w3c, b3, n, nw, nb):
    dn = (((1,), (1,)), ((), ()))

    def body(rad_r, e0, w1, b1_, w2, b2_, w3_, w3c_, b3_, ae_o, tot_o, q_o):
        rad = rad_r[...]
        w1a = w1[0:128, :]
        w1r = w1[128:136, :]
        z1a = jnp.dot(e0[...], w1a, preferred_element_type=jnp.float32) + b1_[...]
        z1 = jnp.dot(rad, w1r, preferred_element_type=jnp.float32) + z1a
        s1 = 1.0 / (1.0 + jnp.exp(-z1))
        h1 = z1 * s1
        z2 = jnp.dot(h1, w2[...], preferred_element_type=jnp.float32) + b2_[...]
        s2 = 1.0 / (1.0 + jnp.exp(-z2))
        h2 = z2 * s2
        ae = jnp.dot(h2, w3c_[...],
                     preferred_element_type=jnp.float32) + b3_[0, 0]
        ae_o[...] = ae
        psum = jnp.sum(ae)

        @pl.when(pl.program_id(0) == 0)
        def _init():
            tot_o[0, 0] = psum

        @pl.when(pl.program_id(0) > 0)
        def _acc():
            tot_o[0, 0] = tot_o[0, 0] + psum

        sp2 = s2 * (1.0 + z2 * (1.0 - s2))
        t2 = sp2 * w3_[...]
        dh1 = lax.dot_general(t2, w2[...], dn,
                              preferred_element_type=jnp.float32)
        sp1 = s1 * (1.0 + z1 * (1.0 - s1))
        dz1 = dh1 * sp1
        q_o[...] = lax.dot_general(dz1, w1r, dn,
                                   preferred_element_type=jnp.float32)

    f32 = jnp.float32
    full = lambda shape: pl.BlockSpec(shape, lambda g: (0,) * len(shape))
    return pl.pallas_call(
        body,
        grid=(n // nb,),
        in_specs=[
            pl.BlockSpec((nb, NUM_RS), lambda g: (g, 0)),
            full((1, 128)),
            full((136, 128)),
            full((1, 128)),
            full((128, 128)),
            full((1, 128)),
            full((1, 128)),
            full((128, 1)),
            pl.BlockSpec(memory_space=pltpu.SMEM),
        ],
        out_specs=[
            pl.BlockSpec((nb, 1), lambda g: (g, 0)),
            pl.BlockSpec(memory_space=pltpu.SMEM),
            pl.BlockSpec((nb, NUM_RS), lambda g: (g, 0)),
        ],
        out_shape=[
            jax.ShapeDtypeStruct((n, 1), f32),
            jax.ShapeDtypeStruct((1, 1), f32),
            jax.ShapeDtypeStruct((n, NUM_RS), f32),
        ],
    )(rad, emb0, W1, b1, W2, b2, w3r, w3c, b3)


def _tc_reduce(f_parts):
    def body(fp, o):
        o[...] = jnp.sum(fp[...], axis=0)

    nw, three, n = f_parts.shape
    return pl.pallas_call(
        body,
        out_shape=jax.ShapeDtypeStruct((three, n), jnp.float32),
    )(f_parts)


def kernel(atomic_numbers, positions, edge_index, atom_emb, W1, b1, W2, b2,
           W3, b3):
    n = positions.shape[0]
    e = edge_index.shape[1]
    info = plsc.get_sparse_core_info()
    nw = info.num_cores * info.num_subcores
    chunk = 2000
    cap = 1024
    pos_flat = positions.reshape(-1)
    src = edge_index[0]
    dst = edge_index[1]
    rad_parts, cmp, cnts = _sc_forward(pos_flat, src, dst, n, e, nw, chunk,
                                       cap)
    rad = _tc_reduce2d(rad_parts).reshape(n, NUM_RS)
    ae_col, tot, q_ne = _tc_dense(
        rad, atom_emb[0:1], W1,
        b1.reshape(1, -1), W2, b2.reshape(1, -1), W3.reshape(1, -1), W3,
        b3.reshape(1, 1), n, nw, 2000)
    f_parts = _sc_backward(cmp, cnts, q_ne.reshape(-1), n, nw,
                           (e // nw) // chunk, cap)
    forces3 = _tc_reduce(f_parts.reshape(nw, 3, n))
    return (tot[0, 0], forces3.T, ae_col.reshape(-1))
